# baseline (speedup 1.0000x reference)
import jax
import jax.numpy as jnp
from jax.experimental import pallas as pl
from jax.experimental.pallas import tpu as pltpu

_OPF = 140
_EMB = 128
_CFG = 18
_OUT = 128


def _main_kernel(opf_ref, cfg_ref, idx_ref, wop_ref, wemb_ref, wcfg_ref,
                 emb_ref, cfgw_ref, opw_ref, b_ref, out_ref, proj_scr):
    blk = opf_ref.shape[0]

    @pl.when(pl.program_id(0) == 0)
    def _prep():
        emb = emb_ref[...]
        norm = jnp.sqrt(jnp.sum(emb * emb, axis=1, keepdims=True))
        scale = jnp.minimum(1.0, 1.0 / jnp.maximum(norm, 1e-7)) * opw_ref[0, 0]
        proj_scr[...] = (
            jnp.dot(emb * scale, wemb_ref[...],
                    preferred_element_type=jnp.float32)
            + b_ref[...]
        )

    idx = idx_ref[...].reshape(1, blk)
    oh_t = (jax.lax.broadcasted_iota(jnp.int32, (_EMB, blk), 0) == idx
            ).astype(jnp.float32)
    acc = jnp.dot(opf_ref[...], wop_ref[...],
                  preferred_element_type=jnp.float32)
    acc += jnp.dot(cfg_ref[...] * cfgw_ref[...], wcfg_ref[...],
                   preferred_element_type=jnp.float32)
    acc += jax.lax.dot_general(oh_t, proj_scr[...],
                               (((0,), (0,)), ((), ())),
                               preferred_element_type=jnp.float32)
    out_ref[...] = acc


def kernel(op_feats, config_feats, emb_table, op_weights, config_weights,
           W, b, op_code):
    n = op_feats.shape[0]
    w_op = W[0:_OPF]
    w_emb = W[_OPF:_OPF + _EMB]
    w_cfg = W[_OPF + _EMB:]
    b2 = b.reshape(1, _OUT)

    blk = 12288
    grid = -(-n // blk)
    idx = op_code.astype(jnp.int32)

    out = pl.pallas_call(
        _main_kernel,
        grid=(grid,),
        in_specs=[
            pl.BlockSpec((blk, _OPF), lambda i: (i, 0)),
            pl.BlockSpec((blk, _CFG), lambda i: (i, 0)),
            pl.BlockSpec((blk,), lambda i: (i,)),
            pl.BlockSpec((_OPF, _OUT), lambda i: (0, 0)),
            pl.BlockSpec((_EMB, _OUT), lambda i: (0, 0)),
            pl.BlockSpec((_CFG, _OUT), lambda i: (0, 0)),
            pl.BlockSpec((_EMB, _EMB), lambda i: (0, 0)),
            pl.BlockSpec((1, _CFG), lambda i: (0, 0)),
            pl.BlockSpec((1, 1), lambda i: (0, 0)),
            pl.BlockSpec((1, _OUT), lambda i: (0, 0)),
        ],
        out_specs=pl.BlockSpec((blk, _OUT), lambda i: (i, 0)),
        out_shape=jax.ShapeDtypeStruct((n, _OUT), jnp.float32),
        scratch_shapes=[pltpu.VMEM((_EMB, _OUT), jnp.float32)],
        compiler_params=pltpu.CompilerParams(
            dimension_semantics=("arbitrary",)),
    )(op_feats, config_feats, idx, w_op, w_emb, w_cfg, emb_table,
      config_weights, op_weights, b2)
    return out


# speedup vs baseline: 1.0101x; 1.0101x over previous
import jax
import jax.numpy as jnp
from jax.experimental import pallas as pl
from jax.experimental.pallas import tpu as pltpu

_OPF = 140
_EMB = 128
_CFG = 18
_OUT = 128


def _main_kernel(opf_ref, cfg_ref, idx_ref, wop_ref, wemb_ref, wcfg_ref,
                 emb_ref, cfgw_ref, opw_ref, b_ref, out_ref, proj_scr):
    blk = opf_ref.shape[0]

    @pl.when(pl.program_id(0) == 0)
    def _prep():
        emb = emb_ref[...]
        norm = jnp.sqrt(jnp.sum(emb * emb, axis=1, keepdims=True))
        scale = jnp.minimum(1.0, 1.0 / jnp.maximum(norm, 1e-7)) * opw_ref[0, 0]
        proj_scr[...] = (
            jnp.dot(emb * scale, wemb_ref[...],
                    preferred_element_type=jnp.float32)
            + b_ref[...]
        )

    idx = idx_ref[...].reshape(1, blk)
    oh_t = (jax.lax.broadcasted_iota(jnp.int32, (_EMB, blk), 0) == idx
            ).astype(jnp.float32)
    acc = jnp.dot(opf_ref[...], wop_ref[...],
                  preferred_element_type=jnp.float32)
    acc += jnp.dot(cfg_ref[...] * cfgw_ref[...], wcfg_ref[...],
                   preferred_element_type=jnp.float32)
    acc += jax.lax.dot_general(oh_t, proj_scr[...],
                               (((0,), (0,)), ((), ())),
                               preferred_element_type=jnp.float32)
    out_ref[...] = acc


def kernel(op_feats, config_feats, emb_table, op_weights, config_weights,
           W, b, op_code):
    n = op_feats.shape[0]
    w_op = W[0:_OPF]
    w_emb = W[_OPF:_OPF + _EMB]
    w_cfg = W[_OPF + _EMB:]
    b2 = b.reshape(1, _OUT)

    blk = 10240
    grid = -(-n // blk)
    idx = op_code.astype(jnp.int32)

    out = pl.pallas_call(
        _main_kernel,
        grid=(grid,),
        in_specs=[
            pl.BlockSpec((blk, _OPF), lambda i: (i, 0)),
            pl.BlockSpec((blk, _CFG), lambda i: (i, 0)),
            pl.BlockSpec((blk,), lambda i: (i,)),
            pl.BlockSpec((_OPF, _OUT), lambda i: (0, 0)),
            pl.BlockSpec((_EMB, _OUT), lambda i: (0, 0)),
            pl.BlockSpec((_CFG, _OUT), lambda i: (0, 0)),
            pl.BlockSpec((_EMB, _EMB), lambda i: (0, 0)),
            pl.BlockSpec((1, _CFG), lambda i: (0, 0)),
            pl.BlockSpec((1, 1), lambda i: (0, 0)),
            pl.BlockSpec((1, _OUT), lambda i: (0, 0)),
        ],
        out_specs=pl.BlockSpec((blk, _OUT), lambda i: (i, 0)),
        out_shape=jax.ShapeDtypeStruct((n, _OUT), jnp.float32),
        scratch_shapes=[pltpu.VMEM((_EMB, _OUT), jnp.float32)],
        compiler_params=pltpu.CompilerParams(
            dimension_semantics=("arbitrary",)),
    )(op_feats, config_feats, idx, w_op, w_emb, w_cfg, emb_table,
      config_weights, op_weights, b2)
    return out
